# baseline (speedup 1.0000x reference)
import functools

import jax
import jax.numpy as jnp
from jax import lax
from jax.experimental import pallas as pl
from jax.experimental.pallas import tpu as pltpu
from jax.experimental.pallas import tpu_sc as plsc

B = 4096
CTX = 20
D = 64
NC = 2
NS = 16
NW = NC * NS
BW = B // NW
LANES = 16


def _sc_gather_mean(x_hbm, emb_hbm, m_hbm, idx_v, rows_v, acc_v, sem0, sem1):
    wid = lax.axis_index("s") * NC + lax.axis_index("c")
    pltpu.sync_copy(x_hbm.at[wid], idx_v)
    sems = (sem0, sem1)
    copies = [None] * CTX
    copies[0] = pltpu.async_copy(emb_hbm.at[idx_v.at[0]], rows_v.at[0], sems[0])
    inv = jnp.float32(1.0 / CTX)
    for j in range(CTX):
        buf = j % 2
        if j + 1 < CTX:
            nbuf = (j + 1) % 2
            copies[j + 1] = pltpu.async_copy(
                emb_hbm.at[idx_v.at[j + 1]], rows_v.at[nbuf], sems[nbuf])
        copies[j].wait()

        @pl.loop(0, BW)
        def _(bb, j=j, buf=buf):
            for c in range(D // LANES):
                sl = pl.ds(c * LANES, LANES)
                r = rows_v[buf, bb, sl]
                if j == 0:
                    acc_v[bb, sl] = r
                elif j == CTX - 1:
                    acc_v[bb, sl] = (acc_v[bb, sl] + r) * inv
                else:
                    acc_v[bb, sl] = acc_v[bb, sl] + r

    pltpu.sync_copy(acc_v, m_hbm.at[pl.ds(wid * BW, BW)])


def _pool_context(x, emb):
    xw = x.reshape(NW, BW, CTX).transpose(0, 2, 1)
    mesh = plsc.VectorSubcoreMesh(core_axis_name="c", subcore_axis_name="s")
    run = functools.partial(
        pl.kernel,
        mesh=mesh,
        out_type=jax.ShapeDtypeStruct((B, D), jnp.float32),
        scratch_types=[
            pltpu.VMEM((CTX, BW), jnp.int32),
            pltpu.VMEM((2, BW, D), jnp.float32),
            pltpu.VMEM((BW, D), jnp.float32),
            pltpu.SemaphoreType.DMA,
            pltpu.SemaphoreType.DMA,
        ],
        compiler_params=pltpu.CompilerParams(use_tc_tiling_on_sc=False),
    )(_sc_gather_mean)
    return run(xw, emb)


VB = 1536


def _mm_body(wt_ref, m_ref, b_ref, o_ref):
    o_ref[...] = lax.dot_general(
        wt_ref[...].astype(jnp.bfloat16), m_ref[...].astype(jnp.bfloat16),
        dimension_numbers=(((0,), (1,)), ((), ())),
        preferred_element_type=jnp.float32,
    ) + jnp.transpose(b_ref[...], (1, 0))


def _project(m, W, b):
    V = W.shape[0]
    grid = (pl.cdiv(V, VB),)
    outT = pl.pallas_call(
        _mm_body,
        grid=grid,
        in_specs=[
            pl.BlockSpec((D, VB), lambda v: (0, v)),
            pl.BlockSpec((B, D), lambda v: (0, 0)),
            pl.BlockSpec((1, VB), lambda v: (0, v)),
        ],
        out_specs=pl.BlockSpec((VB, B), lambda v: (v, 0)),
        out_shape=jax.ShapeDtypeStruct((V, B), jnp.float32),
        compiler_params=pltpu.CompilerParams(
            dimension_semantics=("arbitrary",)),
    )(W.T, m, b.reshape(1, V))
    return outT.T


def kernel(x, emb, W, b):
    m = _pool_context(x.astype(jnp.int32), emb)
    return _project(m, W, b)
